# Initial kernel scaffold; baseline (speedup 1.0000x reference)
import jax
import jax.numpy as jnp
from jax.experimental import pallas as pl


def kernel(left_idx, right_idx, neg_right, neg_left, head_rows, head_cols, head_vals, tail_rows, tail_cols, tail_vals, er_rows, er_cols, er_vals, adj_rows, adj_cols, adj_vals, mask, word_emb, kernel_gate, bias_gate, W1, W2, Dense, Bias):
    raise NotImplementedError("write your pallas kernel here")


# speedup vs baseline: 1.0810x; 1.0810x over previous
import jax
import jax.numpy as jnp
from jax.experimental import pallas as pl

E_NODES = 10000
D = 128
N_REL = 1000
GAMMA = 1.0


def _spmm(rows, cols, vals, X, n_rows):
    return jax.ops.segment_sum(vals[:, None] * jnp.take(X, cols, axis=0), rows, num_segments=n_rows)


def _loss_tc(l_x, r_x, neg_r_x, neg_l_x, mask):
    T, K = mask.shape
    TB = 200
    nblk = T // TB

    def body(l_ref, r_ref, nr_ref, nl_ref, m_ref, o_ref):
        i = pl.program_id(0)
        A = jnp.sum(jnp.abs(l_ref[...] - r_ref[...]), axis=1)
        Dm = A + GAMMA
        B = jnp.sum(jnp.abs(jnp.repeat(l_ref[...], K, axis=0) - nr_ref[...]), axis=1)
        L1 = jax.nn.relu(-B.reshape(TB, K) + Dm[:, None]) * m_ref[...]
        B2 = jnp.sum(jnp.abs(nl_ref[...] - jnp.repeat(r_ref[...], K, axis=0)), axis=1)
        L2 = jax.nn.relu(-B2.reshape(TB, K) + Dm[:, None]) * m_ref[...]
        part = ((jnp.sum(L1) + jnp.sum(L2)) / 2.0).reshape(1, 1)

        @pl.when(i == 0)
        def _():
            o_ref[...] = jnp.zeros_like(o_ref)

        o_ref[...] += part

    out = pl.pallas_call(
        body,
        grid=(nblk,),
        in_specs=[
            pl.BlockSpec((TB, D), lambda i: (i, 0)),
            pl.BlockSpec((TB, D), lambda i: (i, 0)),
            pl.BlockSpec((TB * K, D), lambda i: (i, 0)),
            pl.BlockSpec((TB * K, D), lambda i: (i, 0)),
            pl.BlockSpec((TB, K), lambda i: (i, 0)),
        ],
        out_specs=pl.BlockSpec((1, 1), lambda i: (0, 0)),
        out_shape=jax.ShapeDtypeStruct((1, 1), jnp.float32),
    )(l_x, r_x, neg_r_x, neg_l_x, mask)
    return out[0, 0]


def kernel(left_idx, right_idx, neg_right, neg_left, head_rows, head_cols, head_vals, tail_rows, tail_cols, tail_vals, er_rows, er_cols, er_vals, adj_rows, adj_cols, adj_vals, mask, word_emb, kernel_gate, bias_gate, W1, W2, Dense, Bias):
    norm = jnp.maximum(jnp.linalg.norm(word_emb, axis=-1, keepdims=True), 1e-12)
    we = word_emb / norm
    L = _spmm(head_rows, head_cols, head_vals, we, N_REL)
    R = _spmm(tail_rows, tail_cols, tail_vals, we, N_REL)
    Z1 = L @ Dense[D:2 * D] + R @ Dense[2 * D:]
    Z = jnp.concatenate([Z1, -Z1], axis=0)
    neighbor_z = _spmm(er_rows, er_cols, er_vals, Z, E_NODES)
    P = we @ Dense[:D] + Bias
    nr = we + jax.nn.relu(P + neighbor_z)

    def add_dense(inl, W):
        return jax.nn.relu(_spmm(adj_rows, adj_cols, adj_vals, inl @ W, E_NODES))

    def highway(l1, l2):
        tg = jax.nn.sigmoid(l1 @ kernel_gate + bias_gate)
        return tg * l2 + (1.0 - tg) * l1

    g1 = add_dense(nr, W1)
    h1 = highway(nr, g1)
    g2 = add_dense(h1, W2)
    node = highway(h1, g2)

    t, k = neg_right.shape
    l_x = jnp.take(node, left_idx, axis=0)
    r_x = jnp.take(node, right_idx, axis=0)
    neg_r_x = jnp.take(node, neg_right.reshape(t * k), axis=0)
    neg_l_x = jnp.take(node, neg_left.reshape(t * k), axis=0)
    return _loss_tc(l_x, r_x, neg_r_x, neg_l_x, mask)


# speedup vs baseline: 3.6284x; 3.3565x over previous
import functools

import jax
import jax.numpy as jnp
from jax import lax
from jax.experimental import pallas as pl
from jax.experimental.pallas import tpu as pltpu
import jax.experimental.pallas.tpu_sc as plsc

E_NODES = 10000
D = 128
N_REL = 1000
GAMMA = 1.0

NW = 32
CHUNK = 128


def _pad2d(x, fill):
    e = x.shape[0]
    rows_needed = -(-e // (NW * CHUNK)) * (NW * CHUNK)
    if rows_needed != e:
        x = jnp.concatenate([x, jnp.full((rows_needed - e,), fill, x.dtype)])
    return x.reshape(rows_needed // CHUNK, CHUNK)


def _sc_mesh():
    return plsc.VectorSubcoreMesh(core_axis_name="c", subcore_axis_name="s")


def _scale_rows(gbuf, valv):
    def scale(g, _):
        v16 = valv[pl.ds(g * 16, 16)]
        for u in range(16):
            vv = jnp.full((16,), v16[u], jnp.float32)
            for j in range(8):
                sl = (g * 16 + u, pl.ds(j * 16, 16))
                gbuf[sl] = gbuf[sl] * vv
        return 0
    lax.fori_loop(0, CHUNK // 16, scale, 0)


def _spmm_sc(rows2d, cols2d, vals2d, table, n_out, zeros):
    nr = rows2d.shape[0]
    nc = nr // NW

    @functools.partial(
        pl.kernel,
        out_type=jax.ShapeDtypeStruct((2, n_out, D), jnp.float32),
        mesh=_sc_mesh(),
        scratch_types=[
            pltpu.VMEM_SHARED((n_out, D), jnp.float32),
            pltpu.VMEM((CHUNK,), jnp.int32),
            pltpu.VMEM((1, CHUNK), jnp.int32),
            pltpu.VMEM((CHUNK,), jnp.float32),
            pltpu.VMEM((CHUNK, D), jnp.float32),
            pltpu.SemaphoreType.DMA,
        ],
    )
    def k(rows_h, cols_h, vals_h, table_h, zeros_h, out_h, acc, colv, rowv, valv, gbuf, sem):
        c = lax.axis_index("c")
        s = lax.axis_index("s")
        w = c * 16 + s
        nsplit = max(k for k in range(1, 17) if n_out % k == 0 and (n_out // k) % 8 == 0)
        rp = n_out // nsplit

        @pl.when(s < nsplit)
        def _():
            pltpu.sync_copy(zeros_h.at[pl.ds(s * rp, rp)], acc.at[pl.ds(s * rp, rp)])
        plsc.subcore_barrier()

        def chunk(i, _):
            r = w * nc + i
            pltpu.sync_copy(cols_h.at[r], colv)
            pltpu.sync_copy(vals_h.at[r], valv)
            pltpu.sync_copy(rows_h.at[pl.ds(r, 1)], rowv)
            pltpu.async_copy(table_h.at[colv], gbuf, sem).wait()
            _scale_rows(gbuf, valv)
            pltpu.sync_copy(gbuf, acc.at[rowv.at[0]], add=True)
            return 0

        lax.fori_loop(0, nc, chunk, 0)
        plsc.subcore_barrier()

        @pl.when(s == 0)
        def _():
            pltpu.sync_copy(acc, out_h.at[c])

    return k(rows2d, cols2d, vals2d, table, zeros)


def _headtail_sc(hr, hc, hv, tr, tc, tv, we, zeros_rel):
    nr = hr.shape[0]
    nc = nr // NW

    @functools.partial(
        pl.kernel,
        out_type=jax.ShapeDtypeStruct((2, 2, N_REL, D), jnp.float32),
        mesh=_sc_mesh(),
        scratch_types=[
            pltpu.VMEM_SHARED((N_REL, D), jnp.float32),
            pltpu.VMEM_SHARED((N_REL, D), jnp.float32),
            pltpu.VMEM((CHUNK,), jnp.int32),
            pltpu.VMEM((1, CHUNK), jnp.int32),
            pltpu.VMEM((CHUNK,), jnp.float32),
            pltpu.VMEM((CHUNK, D), jnp.float32),
            pltpu.SemaphoreType.DMA,
        ],
    )
    def k(hr_h, hc_h, hv_h, tr_h, tc_h, tv_h, we_h, z_h, out_h,
          accl, accr, colv, rowv, valv, gbuf, sem):
        c = lax.axis_index("c")
        s = lax.axis_index("s")
        w = c * 16 + s

        @pl.when(s == 0)
        def _():
            pltpu.sync_copy(z_h, accl)

        @pl.when(s == 1)
        def _():
            pltpu.sync_copy(z_h, accr)
        plsc.subcore_barrier()

        def make_chunk(rows_h, cols_h, vals_h, acc):
            def chunk(i, _):
                r = w * nc + i
                pltpu.sync_copy(cols_h.at[r], colv)
                pltpu.sync_copy(vals_h.at[r], valv)
                pltpu.sync_copy(rows_h.at[pl.ds(r, 1)], rowv)
                pltpu.async_copy(we_h.at[colv], gbuf, sem).wait()
                _scale_rows(gbuf, valv)
                pltpu.sync_copy(gbuf, acc.at[rowv.at[0]], add=True)
                return 0
            return chunk

        lax.fori_loop(0, nc, make_chunk(hr_h, hc_h, hv_h, accl), 0)
        lax.fori_loop(0, nc, make_chunk(tr_h, tc_h, tv_h, accr), 0)
        plsc.subcore_barrier()

        @pl.when(s == 0)
        def _():
            pltpu.sync_copy(accl, out_h.at[c, 0])

        @pl.when(s == 1)
        def _():
            pltpu.sync_copy(accr, out_h.at[c, 1])

    return k(hr, hc, hv, tr, tc, tv, we, zeros_rel)


def _gather_sc(idx2d, node):
    nr = idx2d.shape[0]
    nc = nr // NW

    @functools.partial(
        pl.kernel,
        out_type=jax.ShapeDtypeStruct((nr * CHUNK, D), jnp.float32),
        mesh=_sc_mesh(),
        scratch_types=[
            pltpu.VMEM((CHUNK,), jnp.int32),
            pltpu.VMEM((CHUNK, D), jnp.float32),
            pltpu.SemaphoreType.DMA,
        ],
    )
    def k(idx_h, node_h, out_h, colv, gbuf, sem):
        c = lax.axis_index("c")
        s = lax.axis_index("s")
        w = c * 16 + s

        def chunk(i, _):
            r = w * nc + i
            pltpu.sync_copy(idx_h.at[r], colv)
            pltpu.async_copy(node_h.at[colv], gbuf, sem).wait()
            pltpu.sync_copy(gbuf, out_h.at[pl.ds(r * CHUNK, CHUNK)])
            return 0

        lax.fori_loop(0, nc, chunk, 0)

    return k(idx2d, node)


def _tc_call(body, out_shapes, *args):
    return pl.pallas_call(
        body,
        out_shape=out_shapes,
    )(*args)


def _tc_norm_p(word_emb, d1, bias):
    def body(w_ref, d1_ref, b_ref, we_ref, p_ref):
        w = w_ref[...]
        norm = jnp.maximum(jnp.sqrt(jnp.sum(w * w, axis=-1, keepdims=True)), 1e-12)
        we = w / norm
        we_ref[...] = we
        p_ref[...] = jnp.dot(we, d1_ref[...], preferred_element_type=jnp.float32) + b_ref[...]

    return _tc_call(
        body,
        (jax.ShapeDtypeStruct((E_NODES, D), jnp.float32),
         jax.ShapeDtypeStruct((E_NODES, D), jnp.float32)),
        word_emb, d1, bias,
    )


def _tc_z(lr_part, d23a, d23b):
    def body(lr_ref, a_ref, b_ref, z_ref):
        l = lr_ref[0, 0] + lr_ref[1, 0]
        r = lr_ref[0, 1] + lr_ref[1, 1]
        z1 = (jnp.dot(l, a_ref[...], preferred_element_type=jnp.float32)
              + jnp.dot(r, b_ref[...], preferred_element_type=jnp.float32))
        z_ref[...] = jnp.concatenate([z1, -z1], axis=0)

    return _tc_call(
        body,
        jax.ShapeDtypeStruct((2 * N_REL, D), jnp.float32),
        lr_part, d23a, d23b,
    )


def _tc_nr(nb_part, we, p, w1, kg, bg):
    def body(nb_ref, we_ref, p_ref, w1_ref, kg_ref, bg_ref, nr_ref, x1_ref, t1_ref):
        neighbor = nb_ref[0] + nb_ref[1]
        nr = we_ref[...] + jax.nn.relu(p_ref[...] + neighbor)
        nr_ref[...] = nr
        x1_ref[...] = jnp.dot(nr, w1_ref[...], preferred_element_type=jnp.float32)
        t1_ref[...] = jax.nn.sigmoid(
            jnp.dot(nr, kg_ref[...], preferred_element_type=jnp.float32) + bg_ref[...])

    return _tc_call(
        body,
        (jax.ShapeDtypeStruct((E_NODES, D), jnp.float32),
         jax.ShapeDtypeStruct((E_NODES, D), jnp.float32),
         jax.ShapeDtypeStruct((E_NODES, D), jnp.float32)),
        nb_part, we, p, w1, kg, bg,
    )


def _tc_h1(s1_part, nr, t1, w2, kg, bg):
    def body(s1_ref, nr_ref, t1_ref, w2_ref, kg_ref, bg_ref, h1_ref, x2_ref, t2_ref):
        g1 = jax.nn.relu(s1_ref[0] + s1_ref[1])
        t1 = t1_ref[...]
        h1 = t1 * g1 + (1.0 - t1) * nr_ref[...]
        h1_ref[...] = h1
        x2_ref[...] = jnp.dot(h1, w2_ref[...], preferred_element_type=jnp.float32)
        t2_ref[...] = jax.nn.sigmoid(
            jnp.dot(h1, kg_ref[...], preferred_element_type=jnp.float32) + bg_ref[...])

    return _tc_call(
        body,
        (jax.ShapeDtypeStruct((E_NODES, D), jnp.float32),
         jax.ShapeDtypeStruct((E_NODES, D), jnp.float32),
         jax.ShapeDtypeStruct((E_NODES, D), jnp.float32)),
        s1_part, nr, t1, w2, kg, bg,
    )


def _tc_node(s2_part, h1, t2):
    def body(s2_ref, h1_ref, t2_ref, node_ref):
        g2 = jax.nn.relu(s2_ref[0] + s2_ref[1])
        t2 = t2_ref[...]
        node_ref[...] = t2 * g2 + (1.0 - t2) * h1_ref[...]

    return _tc_call(
        body,
        jax.ShapeDtypeStruct((E_NODES, D), jnp.float32),
        s2_part, h1, t2,
    )


def _tc_loss(l_x, r_x, neg_r_x, neg_l_x, mask):
    T, K = mask.shape
    TB = 200
    nblk = T // TB

    def body(l_ref, r_ref, nr_ref, nl_ref, m_ref, o_ref):
        i = pl.program_id(0)
        A = jnp.sum(jnp.abs(l_ref[...] - r_ref[...]), axis=1)
        Dm = A + GAMMA
        B = jnp.sum(jnp.abs(jnp.repeat(l_ref[...], K, axis=0) - nr_ref[...]), axis=1)
        L1 = jax.nn.relu(-B.reshape(TB, K) + Dm[:, None]) * m_ref[...]
        B2 = jnp.sum(jnp.abs(nl_ref[...] - jnp.repeat(r_ref[...], K, axis=0)), axis=1)
        L2 = jax.nn.relu(-B2.reshape(TB, K) + Dm[:, None]) * m_ref[...]
        part = ((jnp.sum(L1) + jnp.sum(L2)) / 2.0).reshape(1, 1)

        @pl.when(i == 0)
        def _():
            o_ref[...] = jnp.zeros_like(o_ref)

        o_ref[...] += part

    out = pl.pallas_call(
        body,
        grid=(nblk,),
        in_specs=[
            pl.BlockSpec((TB, D), lambda i: (i, 0)),
            pl.BlockSpec((TB, D), lambda i: (i, 0)),
            pl.BlockSpec((TB * K, D), lambda i: (i, 0)),
            pl.BlockSpec((TB * K, D), lambda i: (i, 0)),
            pl.BlockSpec((TB, K), lambda i: (i, 0)),
        ],
        out_specs=pl.BlockSpec((1, 1), lambda i: (0, 0)),
        out_shape=jax.ShapeDtypeStruct((1, 1), jnp.float32),
    )(l_x, r_x, neg_r_x, neg_l_x, mask)
    return out[0, 0]


def kernel(left_idx, right_idx, neg_right, neg_left, head_rows, head_cols, head_vals, tail_rows, tail_cols, tail_vals, er_rows, er_cols, er_vals, adj_rows, adj_cols, adj_vals, mask, word_emb, kernel_gate, bias_gate, W1, W2, Dense, Bias):
    f32 = jnp.float32
    i32 = jnp.int32
    zeros_nodes = jnp.zeros((E_NODES, D), f32)
    zeros_rel = jnp.zeros((N_REL, D), f32)

    hr, hc, hv = _pad2d(head_rows.astype(i32), 0), _pad2d(head_cols.astype(i32), 0), _pad2d(head_vals, 0.0)
    tr, tc, tv = _pad2d(tail_rows.astype(i32), 0), _pad2d(tail_cols.astype(i32), 0), _pad2d(tail_vals, 0.0)
    err, erc, erv = _pad2d(er_rows.astype(i32), 0), _pad2d(er_cols.astype(i32), 0), _pad2d(er_vals, 0.0)
    ar, ac, av = _pad2d(adj_rows.astype(i32), 0), _pad2d(adj_cols.astype(i32), 0), _pad2d(adj_vals, 0.0)

    we, p = _tc_norm_p(word_emb, Dense[:D], Bias.reshape(1, D))

    lr_part = _headtail_sc(hr, hc, hv, tr, tc, tv, we, zeros_rel)
    z = _tc_z(lr_part, Dense[D:2 * D], Dense[2 * D:])

    nb_part = _spmm_sc(err, erc, erv, z, E_NODES, zeros_nodes)
    nr, x1, t1 = _tc_nr(nb_part, we, p, W1, kernel_gate, bias_gate.reshape(1, D))

    s1_part = _spmm_sc(ar, ac, av, x1, E_NODES, zeros_nodes)
    h1, x2, t2 = _tc_h1(s1_part, nr, t1, W2, kernel_gate, bias_gate.reshape(1, D))

    s2_part = _spmm_sc(ar, ac, av, x2, E_NODES, zeros_nodes)
    node = _tc_node(s2_part, h1, t2)

    t_pairs, k_neg = neg_right.shape
    all_idx = jnp.concatenate([
        left_idx.astype(i32), right_idx.astype(i32),
        neg_right.astype(i32).reshape(-1), neg_left.astype(i32).reshape(-1),
    ])
    idx2d = _pad2d(all_idx, 0)
    g = _gather_sc(idx2d, node)
    l_x = g[:t_pairs]
    r_x = g[t_pairs:2 * t_pairs]
    neg_r_x = g[2 * t_pairs:2 * t_pairs + t_pairs * k_neg]
    neg_l_x = g[2 * t_pairs + t_pairs * k_neg:2 * t_pairs + 2 * t_pairs * k_neg]
    return _tc_loss(l_x, r_x, neg_r_x, neg_l_x, mask)
